# baseline (speedup 1.0000x reference)
import functools

import jax
import jax.numpy as jnp
from jax import lax
from jax.experimental import pallas as pl
from jax.experimental.pallas import tpu as pltpu
from jax.experimental.pallas import tpu_sc as plsc

NC = 2
NS = 16
LANES = 16


def _encode_body(x_ref, w_ref, b_ref, o_ref):
    o_ref[...] = (
        jnp.dot(x_ref[...], w_ref[...], preferred_element_type=jnp.float32)
        + b_ref[...]
    )


def _encode(x, w, b):
    n, d = x.shape
    return pl.pallas_call(
        _encode_body,
        out_shape=jax.ShapeDtypeStruct((n, d), jnp.float32),
    )(x, w, b)


def _edge_emb_body(a_ref, w_ref, b_ref, o_ref):
    o_ref[...] = (
        jnp.dot(a_ref[...], w_ref[...], preferred_element_type=jnp.float32)
        + b_ref[...]
    )


def _edge_emb(edge_attr, w_edge_l, b_edge_l, block_e):
    de, d = w_edge_l.shape
    e = edge_attr.shape[0]
    return pl.pallas_call(
        _edge_emb_body,
        grid=(e // block_e,),
        in_specs=[
            pl.BlockSpec((block_e, de), lambda i: (i, 0)),
            pl.BlockSpec((de, d), lambda i: (0, 0)),
            pl.BlockSpec((1, d), lambda i: (0, 0)),
        ],
        out_specs=pl.BlockSpec((block_e, d), lambda i: (i, 0)),
        out_shape=jax.ShapeDtypeStruct((e, d), jnp.float32),
    )(edge_attr, w_edge_l, b_edge_l.reshape(1, d))


def _layer_body(h_ref, agg_ref, w1_ref, b1_ref, w2_ref, b2_ref, g_ref,
                bt_ref, eps_ref, o_ref):
    h = h_ref[...]
    agg = agg_ref[0] + agg_ref[1]
    z = h * (1.0 + eps_ref[0, 0]) + agg
    t = jnp.maximum(
        jnp.dot(z, w1_ref[...], preferred_element_type=jnp.float32)
        + b1_ref[...], 0.0)
    t = jnp.dot(t, w2_ref[...], preferred_element_type=jnp.float32) + b2_ref[...]
    mean = jnp.mean(t, axis=0, keepdims=True)
    var = jnp.mean((t - mean) * (t - mean), axis=0, keepdims=True)
    o_ref[...] = (t - mean) * lax.rsqrt(var + 1e-5) * g_ref[...] + bt_ref[...] + h


def _layer(h, agg, w1, b1, w2, b2, gamma, beta, eps_l):
    n, d = h.shape
    return pl.pallas_call(
        _layer_body,
        out_shape=jax.ShapeDtypeStruct((n, d), jnp.float32),
    )(h, agg, w1, b1, w2, b2, gamma, beta, eps_l)


def _sc_message_pass(h, emb, src, dst, chunk=40, nbuf=4):
    n, d = h.shape
    e = src.shape[0]
    nw = NC * NS
    ew = e // nw
    nchunk = ew // chunk
    assert ew % chunk == 0 and chunk % 8 == 0 and chunk <= 128
    npt = (n // NS) // 8 * 8
    tail = n - npt * NS
    zrows = 48
    assert npt % zrows == 0 and tail % 8 == 0 and tail <= zrows
    groups = d // LANES

    mesh = plsc.VectorSubcoreMesh(core_axis_name="c", subcore_axis_name="s")

    @functools.partial(
        pl.kernel,
        out_type=jax.ShapeDtypeStruct((NC, n, d), jnp.float32),
        mesh=mesh,
        scratch_types=[
            [pltpu.VMEM((chunk,), jnp.int32)] * nbuf,
            [pltpu.VMEM((chunk,), jnp.int32)] * nbuf,
            [pltpu.VMEM((chunk, d), jnp.float32)] * nbuf,
            [pltpu.VMEM((chunk, d), jnp.float32)] * nbuf,
            pltpu.VMEM((zrows, d), jnp.float32),
            pltpu.VMEM_SHARED((n, d), jnp.float32),
            [pltpu.SemaphoreType.DMA] * nbuf,
            [pltpu.SemaphoreType.DMA] * nbuf,
            [pltpu.SemaphoreType.DMA] * nbuf,
        ],
    )
    def body(h_hbm, emb_hbm, src_hbm, dst_hbm, out_hbm,
             srcv, dstv, rows, embv, zbuf, aggs, isem, gsem, ssem):
        cid = lax.axis_index("c")
        sid = lax.axis_index("s")
        wid = sid * NC + cid

        @plsc.parallel_loop(0, zrows, 1, unroll=4)
        def _(i):
            for j in range(groups):
                zbuf[i, pl.ds(j * LANES, LANES)] = jnp.zeros(
                    (LANES,), jnp.float32)
        for k in range(npt // zrows):
            pltpu.sync_copy(zbuf, aggs.at[pl.ds(sid * npt + k * zrows, zrows)])
        if tail:
            @pl.when(sid == 0)
            def _():
                pltpu.sync_copy(zbuf.at[pl.ds(0, tail)],
                                aggs.at[pl.ds(NS * npt, tail)])
        plsc.subcore_barrier()

        def idx_loads(c, b):
            base = wid * ew + c * chunk
            pltpu.async_copy(src_hbm.at[pl.ds(base, chunk)], srcv[b], isem[b])
            pltpu.async_copy(dst_hbm.at[pl.ds(base, chunk)], dstv[b], isem[b])

        def data_loads(c, b):
            base = wid * ew + c * chunk
            pltpu.make_async_copy(src_hbm.at[pl.ds(base, chunk)], srcv[b],
                                  isem[b]).wait()
            pltpu.make_async_copy(dst_hbm.at[pl.ds(base, chunk)], dstv[b],
                                  isem[b]).wait()
            pltpu.async_copy(h_hbm.at[srcv[b]], rows[b], gsem[b])
            pltpu.async_copy(emb_hbm.at[pl.ds(base, chunk), :], embv[b],
                             gsem[b])

        def wait_scatter(b):
            pltpu.make_async_copy(rows[b], aggs.at[dstv[b]], ssem[b]).wait()

        def process(c, b):
            base = wid * ew + c * chunk
            pltpu.make_async_copy(h_hbm.at[srcv[b]], rows[b], gsem[b]).wait()
            pltpu.make_async_copy(emb_hbm.at[pl.ds(base, chunk), :], embv[b],
                                  gsem[b]).wait()

            @plsc.parallel_loop(0, chunk, 1, unroll=8)
            def _(i):
                for j in range(groups):
                    sl = pl.ds(j * LANES, LANES)
                    rows[b][i, sl] = jnp.maximum(
                        rows[b][i, sl] + embv[b][i, sl], 0.0)
            pltpu.async_copy(rows[b], aggs.at[dstv[b]], ssem[b], add=True)

        idx_loads(0, 0)
        idx_loads(1, 1)
        data_loads(0, 0)

        nmain = (nchunk - 4) // nbuf

        def round_body(g, carry):
            for b in range(nbuf):
                c = g * nbuf + b
                process(c, b)

                @pl.when(c >= 2)
                def _():
                    wait_scatter((b + 2) % nbuf)
                idx_loads(c + 2, (b + 2) % nbuf)
                data_loads(c + 1, (b + 1) % nbuf)
            return carry
        lax.fori_loop(0, nmain, round_body, 0)

        for c in range(nmain * nbuf, nchunk):
            process(c, c % nbuf)
            if c >= 2:
                wait_scatter((c - 2) % nbuf)
            if c + 2 < nchunk:
                idx_loads(c + 2, (c + 2) % nbuf)
            if c + 1 < nchunk:
                data_loads(c + 1, (c + 1) % nbuf)
        for c in range(max(nchunk - 2, 0), nchunk):
            wait_scatter(c % nbuf)

        plsc.subcore_barrier()
        for k in range(npt // zrows):
            r0 = sid * npt + k * zrows
            pltpu.sync_copy(aggs.at[pl.ds(r0, zrows)],
                            out_hbm.at[cid, pl.ds(r0, zrows), :])
        if tail:
            @pl.when(sid == 0)
            def _():
                pltpu.sync_copy(aggs.at[pl.ds(NS * npt, tail)],
                                out_hbm.at[cid, pl.ds(NS * npt, tail), :])

    return body(h, emb, src, dst)


def kernel(x, edge_index, edge_attr, W_init, b_init, W_edge, b_edge, eps,
           W1, b1, W2, b2, gamma, beta):
    num_l = W_edge.shape[0]
    src = edge_index[0]
    dst = edge_index[1]

    h = _encode(x, W_init, b_init)

    for l in range(num_l):
        emb_l = _edge_emb(edge_attr, W_edge[l], b_edge[l], block_e=8000)
        agg = _sc_message_pass(h, emb_l, src, dst)
        eps_l = eps[l].reshape(1, 1)
        h = _layer(h, agg, W1[l], b1[l], W2[l], b2[l],
                   gamma[l].reshape(1, -1), beta[l].reshape(1, -1), eps_l)
    return h


# speedup vs baseline: 1.2249x; 1.2249x over previous
import functools

import numpy as np

import jax
import jax.numpy as jnp
from jax import lax
from jax.experimental import pallas as pl
from jax.experimental.pallas import tpu as pltpu
from jax.experimental.pallas import tpu_sc as plsc

NC = 2
NS = 16
LANES = 16

_COLS_E = np.arange(128).reshape(4, 2, 16)[:, 0, :].reshape(64)
_COLS_O = np.arange(128).reshape(4, 2, 16)[:, 1, :].reshape(64)


def _pack_bf16(ue, uo):
    ie = lax.bitcast_convert_type(
        ue.astype(jnp.bfloat16).astype(jnp.float32), jnp.uint32)
    io = lax.bitcast_convert_type(
        uo.astype(jnp.bfloat16).astype(jnp.float32), jnp.uint32)
    return lax.bitcast_convert_type((ie >> 16) | io, jnp.int32)


def _encode_body(x_ref, w_ref, b_ref, we_ref, be_ref, wo_ref, bo_ref,
                 o_ref, ob_ref):
    x = x_ref[...]
    o_ref[...] = (
        jnp.dot(x, w_ref[...], preferred_element_type=jnp.float32)
        + b_ref[...]
    )
    ue = jnp.dot(x, we_ref[...], preferred_element_type=jnp.float32) + be_ref[...]
    uo = jnp.dot(x, wo_ref[...], preferred_element_type=jnp.float32) + bo_ref[...]
    ob_ref[...] = _pack_bf16(ue, uo)


def _encode(x, w, b, we, be, wo, bo):
    n, d = x.shape
    return pl.pallas_call(
        _encode_body,
        out_shape=[jax.ShapeDtypeStruct((n, d), jnp.float32),
                   jax.ShapeDtypeStruct((n, d // 2), jnp.int32)],
    )(x, w, b, we, be, wo, bo)


def _edge_emb_body(a_ref, we_ref, be_ref, wo_ref, bo_ref, o_ref):
    a = a_ref[...]
    ue = jnp.dot(a, we_ref[...], preferred_element_type=jnp.float32) + be_ref[...]
    uo = jnp.dot(a, wo_ref[...], preferred_element_type=jnp.float32) + bo_ref[...]
    o_ref[...] = _pack_bf16(ue, uo)


def _edge_emb(edge_attr, we, be, wo, bo, block_e):
    de, dh = we.shape
    e = edge_attr.shape[0]
    return pl.pallas_call(
        _edge_emb_body,
        grid=(e // block_e,),
        in_specs=[
            pl.BlockSpec((block_e, de), lambda i: (i, 0)),
            pl.BlockSpec((de, dh), lambda i: (0, 0)),
            pl.BlockSpec((1, dh), lambda i: (0, 0)),
            pl.BlockSpec((de, dh), lambda i: (0, 0)),
            pl.BlockSpec((1, dh), lambda i: (0, 0)),
        ],
        out_specs=pl.BlockSpec((block_e, dh), lambda i: (i, 0)),
        out_shape=jax.ShapeDtypeStruct((e, dh), jnp.int32),
    )(edge_attr, we, be.reshape(1, dh), wo, bo.reshape(1, dh))


def _layer_body(h_ref, hb_ref, agg_ref, w1_ref, b1_ref, w2_ref, b2_ref,
                w2e_ref, b2e_ref, w2o_ref, b2o_ref, g_ref, bt_ref,
                ge_ref, bte_ref, go_ref, bto_ref, eps_ref, o_ref, ob_ref):
    h = h_ref[...]
    agg = agg_ref[0] + agg_ref[1]
    z = h * (1.0 + eps_ref[0, 0]) + agg
    t = jnp.maximum(
        jnp.dot(z, w1_ref[...], preferred_element_type=jnp.float32)
        + b1_ref[...], 0.0)
    u = jnp.dot(t, w2_ref[...], preferred_element_type=jnp.float32) + b2_ref[...]
    mean = jnp.mean(u, axis=0, keepdims=True)
    var = jnp.mean((u - mean) * (u - mean), axis=0, keepdims=True)
    o_ref[...] = (u - mean) * lax.rsqrt(var + 1e-5) * g_ref[...] + bt_ref[...] + h

    w = lax.bitcast_convert_type(hb_ref[...], jnp.uint32)
    he = lax.bitcast_convert_type(w << 16, jnp.float32)
    ho = lax.bitcast_convert_type(w & jnp.uint32(0xFFFF0000), jnp.float32)
    ue = jnp.dot(t, w2e_ref[...], preferred_element_type=jnp.float32) + b2e_ref[...]
    uo = jnp.dot(t, w2o_ref[...], preferred_element_type=jnp.float32) + b2o_ref[...]
    meane = jnp.mean(ue, axis=0, keepdims=True)
    vare = jnp.mean((ue - meane) * (ue - meane), axis=0, keepdims=True)
    meano = jnp.mean(uo, axis=0, keepdims=True)
    varo = jnp.mean((uo - meano) * (uo - meano), axis=0, keepdims=True)
    ve = (ue - meane) * lax.rsqrt(vare + 1e-5) * ge_ref[...] + bte_ref[...] + he
    vo = (uo - meano) * lax.rsqrt(varo + 1e-5) * go_ref[...] + bto_ref[...] + ho
    ob_ref[...] = _pack_bf16(ve, vo)


def _layer(h, hb, agg, w1, b1, w2, b2, w2e, b2e, w2o, b2o, gamma, beta,
           ge, bte, go, bto, eps_l):
    n, d = h.shape
    return pl.pallas_call(
        _layer_body,
        out_shape=[jax.ShapeDtypeStruct((n, d), jnp.float32),
                   jax.ShapeDtypeStruct((n, d // 2), jnp.int32)],
    )(h, hb, agg, w1, b1, w2, b2, w2e, b2e, w2o, b2o, gamma, beta,
      ge, bte, go, bto, eps_l)


def _sc_message_pass(hb, embb, src, dst, n, d, chunk=80):
    dh = d // 2
    e = src.shape[0]
    nw = NC * NS
    ew = e // nw
    nchunk = ew // chunk
    assert ew % chunk == 0 and chunk % 8 == 0 and chunk <= 128
    nbuf = 2
    nidx = 4
    npt = (n // NS) // 8 * 8
    tail = n - npt * NS
    zrows = 48
    assert npt % zrows == 0 and tail % 8 == 0 and tail <= zrows
    pairs = d // (2 * LANES)

    mesh = plsc.VectorSubcoreMesh(core_axis_name="c", subcore_axis_name="s")

    @functools.partial(
        pl.kernel,
        out_type=jax.ShapeDtypeStruct((NC, n, d), jnp.float32),
        mesh=mesh,
        compiler_params=pltpu.CompilerParams(use_tc_tiling_on_sc=False),
        scratch_types=[
            [pltpu.VMEM((chunk,), jnp.int32)] * nidx,
            [pltpu.VMEM((chunk,), jnp.int32)] * nidx,
            [pltpu.VMEM((chunk, dh), jnp.int32)] * nbuf,
            [pltpu.VMEM((chunk, dh), jnp.int32)] * nbuf,
            [pltpu.VMEM((chunk, d), jnp.float32)] * nbuf,
            pltpu.VMEM((zrows, d), jnp.float32),
            pltpu.VMEM_SHARED((n, d), jnp.float32),
            [pltpu.SemaphoreType.DMA] * nidx,
            [pltpu.SemaphoreType.DMA] * nbuf,
            [pltpu.SemaphoreType.DMA] * nbuf,
        ],
    )
    def body(h_hbm, emb_hbm, src_hbm, dst_hbm, out_hbm,
             srcv, dstv, rows, embv, msg, zbuf, aggs, isem, gsem, ssem):
        cid = lax.axis_index("c")
        sid = lax.axis_index("s")
        wid = sid * NC + cid

        @plsc.parallel_loop(0, zrows, 1, unroll=4)
        def _(i):
            for j in range(d // LANES):
                zbuf[i, pl.ds(j * LANES, LANES)] = jnp.zeros(
                    (LANES,), jnp.float32)
        for k in range(npt // zrows):
            pltpu.sync_copy(zbuf, aggs.at[pl.ds(sid * npt + k * zrows, zrows)])
        if tail:
            @pl.when(sid == 0)
            def _():
                pltpu.sync_copy(zbuf.at[pl.ds(0, tail)],
                                aggs.at[pl.ds(NS * npt, tail)])
        plsc.subcore_barrier()

        def idx_loads(c, i4):
            base = wid * ew + c * chunk
            pltpu.async_copy(src_hbm.at[pl.ds(base, chunk)], srcv[i4],
                             isem[i4])
            pltpu.async_copy(dst_hbm.at[pl.ds(base, chunk)], dstv[i4],
                             isem[i4])

        def data_loads(c, b, i4):
            base = wid * ew + c * chunk
            pltpu.make_async_copy(src_hbm.at[pl.ds(base, chunk)], srcv[i4],
                                  isem[i4]).wait()
            pltpu.make_async_copy(dst_hbm.at[pl.ds(base, chunk)], dstv[i4],
                                  isem[i4]).wait()
            pltpu.async_copy(h_hbm.at[srcv[i4]], rows[b], gsem[b])
            pltpu.async_copy(emb_hbm.at[pl.ds(base, chunk), :], embv[b],
                             gsem[b])

        def wait_scatter(b, i4):
            pltpu.make_async_copy(msg[b], aggs.at[dstv[i4]], ssem[b]).wait()

        def process(c, b, i4):
            base = wid * ew + c * chunk
            pltpu.make_async_copy(h_hbm.at[srcv[i4]], rows[b], gsem[b]).wait()
            pltpu.make_async_copy(emb_hbm.at[pl.ds(base, chunk), :], embv[b],
                                  gsem[b]).wait()

            @plsc.parallel_loop(0, chunk, 1, unroll=4)
            def _(i):
                for p in range(pairs):
                    sl = pl.ds(LANES * p, LANES)
                    rv = rows[b][i, sl]
                    ev = embv[b][i, sl]
                    r0 = lax.bitcast_convert_type(rv << 16, jnp.float32)
                    e0 = lax.bitcast_convert_type(ev << 16, jnp.float32)
                    r1 = lax.bitcast_convert_type(
                        rv & jnp.int32(-65536), jnp.float32)
                    e1 = lax.bitcast_convert_type(
                        ev & jnp.int32(-65536), jnp.float32)
                    msg[b][i, pl.ds(32 * p, LANES)] = jnp.maximum(
                        r0 + e0, 0.0)
                    msg[b][i, pl.ds(32 * p + LANES, LANES)] = jnp.maximum(
                        r1 + e1, 0.0)
            pltpu.async_copy(msg[b], aggs.at[dstv[i4]], ssem[b], add=True)

        idx_loads(0, 0)
        idx_loads(1, 1)
        data_loads(0, 0, 0)

        nmain = (nchunk - 2) // nidx

        def round_body(g, carry):
            for k in range(nidx):
                c = g * nidx + k
                b = k % nbuf
                process(c, b, k)
                data_loads(c + 1, (k + 1) % nbuf, (k + 1) % nidx)
                idx_loads(c + 2, (k + 2) % nidx)

                @pl.when(c >= 1)
                def _():
                    wait_scatter((k + 1) % nbuf, (k + 3) % nidx)
            return carry
        lax.fori_loop(0, nmain, round_body, 0)

        for c in range(nmain * nidx, nchunk):
            process(c, c % nbuf, c % nidx)
            if c + 1 < nchunk:
                data_loads(c + 1, (c + 1) % nbuf, (c + 1) % nidx)
            if c + 2 < nchunk:
                idx_loads(c + 2, (c + 2) % nidx)
            if c >= 1:
                wait_scatter((c - 1) % nbuf, (c - 1) % nidx)
        wait_scatter((nchunk - 1) % nbuf, (nchunk - 1) % nidx)

        plsc.subcore_barrier()
        for k in range(npt // zrows):
            r0 = sid * npt + k * zrows
            pltpu.sync_copy(aggs.at[pl.ds(r0, zrows)],
                            out_hbm.at[cid, pl.ds(r0, zrows), :])
        if tail:
            @pl.when(sid == 0)
            def _():
                pltpu.sync_copy(aggs.at[pl.ds(NS * npt, tail)],
                                out_hbm.at[cid, pl.ds(NS * npt, tail), :])

    return body(hb, embb, src, dst)


def kernel(x, edge_index, edge_attr, W_init, b_init, W_edge, b_edge, eps,
           W1, b1, W2, b2, gamma, beta):
    num_l = W_edge.shape[0]
    n, d = x.shape
    src = edge_index[0]
    dst = edge_index[1]
    ce = jnp.asarray(_COLS_E)
    co = jnp.asarray(_COLS_O)

    h, hb = _encode(x, W_init, b_init,
                    W_init[:, ce], b_init[ce], W_init[:, co], b_init[co])

    for l in range(num_l):
        emb_l = _edge_emb(edge_attr, W_edge[l][:, ce], b_edge[l][ce],
                          W_edge[l][:, co], b_edge[l][co], block_e=8000)
        agg = _sc_message_pass(hb, emb_l, src, dst, n, d)
        eps_l = eps[l].reshape(1, 1)
        h, hb = _layer(h, hb, agg, W1[l], b1[l], W2[l], b2[l],
                       W2[l][:, ce], b2[l][ce], W2[l][:, co], b2[l][co],
                       gamma[l].reshape(1, -1), beta[l].reshape(1, -1),
                       gamma[l][ce].reshape(1, -1), beta[l][ce].reshape(1, -1),
                       gamma[l][co].reshape(1, -1), beta[l][co].reshape(1, -1),
                       eps_l)
    return h
